# baseline (speedup 1.0000x reference)
import functools

import jax
import jax.numpy as jnp
from jax import lax
from jax.experimental import pallas as pl
from jax.experimental.pallas import tpu as pltpu
from jax.experimental.pallas import tpu_sc as plsc

_B = 16384
_F = 26
_V = 100000
_D = 32

_NC = 2
_NS = 16
_NW = _NC * _NS
_ROWS = _B * _F
_RPW = _ROWS // _NW
_CHUNK = 128
_NCHUNK = _RPW // _CHUNK
_CPF = _B // _CHUNK

_BB = 1024
_H1 = 256
_H2 = 128


def _sc_gather(tables, idx3):
    mesh = plsc.VectorSubcoreMesh(core_axis_name="c", subcore_axis_name="s")

    @functools.partial(
        pl.kernel,
        out_type=jax.ShapeDtypeStruct((_ROWS, _D), jnp.float32),
        mesh=mesh,
        scratch_types=[
            pltpu.VMEM((_NCHUNK, _CHUNK), jnp.int32),
            pltpu.VMEM((2, _CHUNK, _D), jnp.float32),
            pltpu.SemaphoreType.DMA,
            pltpu.SemaphoreType.DMA,
        ],
        compiler_params=pltpu.CompilerParams(use_tc_tiling_on_sc=False),
    )
    def k(tab_hbm, idx_hbm, out_hbm, idx_v, rows_v, gsem, osem):
        wid = lax.axis_index("s") * _NC + lax.axis_index("c")
        cbase = wid * _NCHUNK
        pltpu.sync_copy(idx_hbm.at[wid], idx_v)

        f0 = lax.div(cbase, _CPF)
        first = pltpu.async_copy(
            tab_hbm.at[f0].at[idx_v.at[0]], rows_v.at[0], gsem
        )
        first.wait()

        def body(j, _):
            slot = lax.rem(j, 2)
            nxt = lax.rem(j + 1, 2)
            c = cbase + j

            @pl.when(j + 1 < _NCHUNK)
            def _():
                f = lax.div(c + 1, _CPF)
                pltpu.async_copy(
                    tab_hbm.at[f].at[idx_v.at[j + 1]], rows_v.at[nxt], gsem
                ).wait()

            pltpu.async_copy(
                rows_v.at[slot],
                out_hbm.at[pl.ds(c * _CHUNK, _CHUNK)],
                osem,
            ).wait()
            return 0

        lax.fori_loop(0, _NCHUNK, body, 0)

    return k(tables, idx3)


def _tc_mlp(rows, W1c, b1c, W2b, b2c, W3b, b3c):
    def body(x_ref, w1_ref, b1_ref, w2_ref, b2_ref, w3_ref, b3_ref, o_ref,
             acc_ref):
        f = pl.program_id(1)
        part = jnp.dot(x_ref[...], w1_ref[...],
                       preferred_element_type=jnp.float32)

        @pl.when(f == 0)
        def _():
            acc_ref[...] = part

        @pl.when(f > 0)
        def _():
            acc_ref[...] += part

        @pl.when(f == _F - 1)
        def _():
            h = jnp.maximum(acc_ref[...] + b1_ref[...], 0.0)
            h = jnp.dot(h, w2_ref[...], preferred_element_type=jnp.float32)
            h = jnp.maximum(h + b2_ref[...], 0.0)
            logits = jnp.dot(h, w3_ref[...],
                             preferred_element_type=jnp.float32)
            p = jax.nn.sigmoid(logits + b3_ref[...])
            pc = p[:, 0:1]
            pv = p[:, 1:2]
            o_ref[...] = jnp.concatenate([pc, pc * pv], axis=1)

    nb = _B // _BB
    return pl.pallas_call(
        body,
        grid=(nb, _F),
        in_specs=[
            pl.BlockSpec((_BB, _D), lambda i, f: (f * nb + i, 0)),
            pl.BlockSpec((_D, 2 * _H1), lambda i, f: (f, 0)),
            pl.BlockSpec((1, 2 * _H1), lambda i, f: (0, 0)),
            pl.BlockSpec((2 * _H1, 2 * _H2), lambda i, f: (0, 0)),
            pl.BlockSpec((1, 2 * _H2), lambda i, f: (0, 0)),
            pl.BlockSpec((2 * _H2, 2), lambda i, f: (0, 0)),
            pl.BlockSpec((1, 2), lambda i, f: (0, 0)),
        ],
        out_specs=pl.BlockSpec((_BB, 2), lambda i, f: (i, 0)),
        out_shape=jax.ShapeDtypeStruct((_B, 2), jnp.float32),
        scratch_shapes=[pltpu.VMEM((_BB, 2 * _H1), jnp.float32)],
    )(rows, W1c, b1c, W2b, b2c, W3b, b3c)


def kernel(indices, tables,
           ctr_W1, ctr_b1, ctr_W2, ctr_b2, ctr_W3, ctr_b3,
           cvr_W1, cvr_b1, cvr_W2, cvr_b2, cvr_W3, cvr_b3):
    idx3 = indices.astype(jnp.int32).reshape(_NW, _NCHUNK, _CHUNK)

    rows = _sc_gather(tables, idx3)

    W1c = jnp.concatenate([ctr_W1, cvr_W1], axis=1)
    b1c = jnp.concatenate([ctr_b1, cvr_b1])[None, :]
    z21 = jnp.zeros((_H1, _H2), jnp.float32)
    W2b = jnp.concatenate(
        [jnp.concatenate([ctr_W2, z21], axis=1),
         jnp.concatenate([z21, cvr_W2], axis=1)], axis=0)
    b2c = jnp.concatenate([ctr_b2, cvr_b2])[None, :]
    z31 = jnp.zeros((_H2, 1), jnp.float32)
    W3b = jnp.concatenate(
        [jnp.concatenate([ctr_W3, z31], axis=1),
         jnp.concatenate([z31, cvr_W3], axis=1)], axis=0)
    b3c = jnp.concatenate([ctr_b3, cvr_b3])[None, :]

    return _tc_mlp(rows, W1c, b1c, W2b, b2c, W3b, b3c)


# speedup vs baseline: 1.3035x; 1.3035x over previous
import functools

import jax
import jax.numpy as jnp
from jax import lax
from jax.experimental import pallas as pl
from jax.experimental.pallas import tpu as pltpu
from jax.experimental.pallas import tpu_sc as plsc

_B = 16384
_F = 26
_V = 100000
_D = 32

_NC = 2
_NS = 16
_NW = _NC * _NS
_ROWS = _B * _F
_RPW = _ROWS // _NW
_CHUNK = 128
_NCHUNK = _RPW // _CHUNK

_BB = 1024
_H1 = 256
_H2 = 128


def _sc_gather(table_flat, idx3):
    mesh = plsc.VectorSubcoreMesh(core_axis_name="c", subcore_axis_name="s")

    @functools.partial(
        pl.kernel,
        out_type=jax.ShapeDtypeStruct((_ROWS, _D), jnp.float32),
        mesh=mesh,
        scratch_types=[
            pltpu.VMEM((_NCHUNK, _CHUNK), jnp.int32),
            pltpu.VMEM((2, _CHUNK, _D), jnp.float32),
            pltpu.SemaphoreType.DMA,
            pltpu.SemaphoreType.DMA,
        ],
        compiler_params=pltpu.CompilerParams(use_tc_tiling_on_sc=False),
    )
    def k(table_hbm, idx_hbm, out_hbm, idx_v, rows_v, gsem, osem):
        wid = lax.axis_index("s") * _NC + lax.axis_index("c")
        base = wid * _RPW
        pltpu.sync_copy(idx_hbm.at[wid], idx_v)

        first = pltpu.async_copy(table_hbm.at[idx_v.at[0]], rows_v.at[0], gsem)
        first.wait()

        def body(j, _):
            slot = lax.rem(j, 2)
            nxt = lax.rem(j + 1, 2)

            @pl.when(j + 1 < _NCHUNK)
            def _():
                pltpu.async_copy(
                    table_hbm.at[idx_v.at[j + 1]], rows_v.at[nxt], gsem
                ).wait()

            pltpu.async_copy(
                rows_v.at[slot],
                out_hbm.at[pl.ds(base + j * _CHUNK, _CHUNK)],
                osem,
            ).wait()
            return 0

        lax.fori_loop(0, _NCHUNK, body, 0)

    return k(table_flat, idx3)


def _tc_mlp(emb, W1c, b1c, W2b, b2c, W3b, b3c):
    def body(x_ref, w1_ref, b1_ref, w2_ref, b2_ref, w3_ref, b3_ref, o_ref):
        x = x_ref[...]
        h = jnp.dot(x, w1_ref[...], preferred_element_type=jnp.float32)
        h = jnp.maximum(h + b1_ref[...], 0.0)
        h = jnp.dot(h, w2_ref[...], preferred_element_type=jnp.float32)
        h = jnp.maximum(h + b2_ref[...], 0.0)
        logits = jnp.dot(h, w3_ref[...], preferred_element_type=jnp.float32)
        p = jax.nn.sigmoid(logits + b3_ref[...])
        pc = p[:, 0:1]
        pv = p[:, 1:2]
        o_ref[...] = jnp.concatenate([pc, pc * pv], axis=1)

    grid = (_B // _BB,)
    return pl.pallas_call(
        body,
        grid=grid,
        in_specs=[
            pl.BlockSpec((_BB, _F * _D), lambda i: (i, 0)),
            pl.BlockSpec((_F * _D, 2 * _H1), lambda i: (0, 0)),
            pl.BlockSpec((1, 2 * _H1), lambda i: (0, 0)),
            pl.BlockSpec((2 * _H1, 2 * _H2), lambda i: (0, 0)),
            pl.BlockSpec((1, 2 * _H2), lambda i: (0, 0)),
            pl.BlockSpec((2 * _H2, 2), lambda i: (0, 0)),
            pl.BlockSpec((1, 2), lambda i: (0, 0)),
        ],
        out_specs=pl.BlockSpec((_BB, 2), lambda i: (i, 0)),
        out_shape=jax.ShapeDtypeStruct((_B, 2), jnp.float32),
    )(emb, W1c, b1c, W2b, b2c, W3b, b3c)


def kernel(indices, tables,
           ctr_W1, ctr_b1, ctr_W2, ctr_b2, ctr_W3, ctr_b3,
           cvr_W1, cvr_b1, cvr_W2, cvr_b2, cvr_W3, cvr_b3):
    idx = indices.astype(jnp.int32)
    flat = idx.T + (jnp.arange(_F, dtype=jnp.int32) * _V)[None, :]
    idx3 = flat.reshape(_NW, _NCHUNK, _CHUNK)

    table_flat = tables.reshape(_F * _V, _D)

    rows = _sc_gather(table_flat, idx3)
    emb = rows.reshape(_B, _F * _D)

    W1c = jnp.concatenate([ctr_W1, cvr_W1], axis=1)
    b1c = jnp.concatenate([ctr_b1, cvr_b1])[None, :]
    z21 = jnp.zeros((_H1, _H2), jnp.float32)
    W2b = jnp.concatenate(
        [jnp.concatenate([ctr_W2, z21], axis=1),
         jnp.concatenate([z21, cvr_W2], axis=1)], axis=0)
    b2c = jnp.concatenate([ctr_b2, cvr_b2])[None, :]
    z31 = jnp.zeros((_H2, 1), jnp.float32)
    W3b = jnp.concatenate(
        [jnp.concatenate([ctr_W3, z31], axis=1),
         jnp.concatenate([z31, cvr_W3], axis=1)], axis=0)
    b3c = jnp.concatenate([ctr_b3, cvr_b3])[None, :]

    return _tc_mlp(emb, W1c, b1c, W2b, b2c, W3b, b3c)
